# baseline (speedup 1.0000x reference)
import functools

import jax
import jax.numpy as jnp
from jax import lax
from jax.experimental import pallas as pl
from jax.experimental.pallas import tpu as pltpu
from jax.experimental.pallas import tpu_sc as plsc

_NE, _NCH = 8192, 32
_N = _NE * _NCH
_NW = 32
_CHUNK = _N // _NW
_L = 16
_NVEC = _CHUNK // _L

_DEAD_LINES = ((0.008, 0.0), (0.032, -0.006), (0.36, -0.17), (1.2, -0.8))
_TAU_LINES = ((0.04, 0.02), (0.08, 0.01), (0.12, -0.01), (0.16, -0.04))


def _maxlines(x, lines):
    (s0, b0), (s1, b1), (s2, b2), (s3, b3) = lines
    return jnp.maximum(
        jnp.maximum(x * s0 + b0, x * s1 + b1),
        jnp.maximum(x * s2 + b2, x * s3 + b3),
    )


def _sc_body(x_hbm, out_hbm, xin, xout):
    wid = lax.axis_index("s") * 2 + lax.axis_index("c")
    base = wid * _CHUNK
    pltpu.sync_copy(x_hbm.at[pl.ds(base, _CHUNK)], xin)

    def step(i, carry):
        t = xin[pl.ds(i * _L, _L)]
        xc = jnp.minimum(jnp.maximum(t, 0.0), 1.0)
        dead = _maxlines(xc, _DEAD_LINES)
        tau = _maxlines(xc, _TAU_LINES)
        delay = dead / 0.005
        read = 102.0 - delay
        w = jnp.minimum(jnp.maximum(read - 101.0, 0.0), 1.0)
        w = jnp.where(read >= 102.0, 0.0, w)
        w = jnp.where(delay == 0.0, 1.0, w)
        af = 0.005 / (tau + 0.005)
        xout[pl.ds(i * _L, _L)] = af * (w * t)
        return carry

    lax.fori_loop(0, _NVEC, step, 0, unroll=8)
    pltpu.sync_copy(xout, out_hbm.at[pl.ds(base, _CHUNK)])


@functools.lru_cache(maxsize=None)
def _build_pam_sc():
    return pl.kernel(
        _sc_body,
        mesh=plsc.VectorSubcoreMesh(core_axis_name="c", subcore_axis_name="s"),
        out_type=jax.ShapeDtypeStruct((_N,), jnp.float32),
        scratch_types=[
            pltpu.VMEM((_CHUNK,), jnp.float32),
            pltpu.VMEM((_CHUNK,), jnp.float32),
        ],
    )


def kernel(target_pressure):
    x = target_pressure.reshape(_N)
    return _build_pam_sc()(x).reshape(_NE, _NCH)


# speedup vs baseline: 1.1909x; 1.1909x over previous
import functools

import jax
import jax.numpy as jnp
from jax import lax
from jax.experimental import pallas as pl
from jax.experimental.pallas import tpu as pltpu
from jax.experimental.pallas import tpu_sc as plsc

_NE, _NCH = 8192, 32
_N = _NE * _NCH
_NW = 32
_CHUNK = _N // _NW
_L = 16
_NVEC = _CHUNK // _L

_DEAD_LINES = ((0.008, 0.0), (0.032, -0.006), (0.36, -0.17), (1.2, -0.8))
_TAU_LINES = ((0.04, 0.02), (0.08, 0.01), (0.12, -0.01), (0.16, -0.04))
_INV_DT = 200.00000447034847


def _maxlines(x, lines):
    (s0, b0), (s1, b1), (s2, b2), (s3, b3) = lines
    return jnp.maximum(
        jnp.maximum(x * s0 + b0, x * s1 + b1),
        jnp.maximum(x * s2 + b2, x * s3 + b3),
    )


def _sc_body(x_hbm, out_hbm, xin, xout):
    wid = lax.axis_index("s") * 2 + lax.axis_index("c")
    base = wid * _CHUNK
    pltpu.sync_copy(x_hbm.at[pl.ds(base, _CHUNK)], xin)

    @plsc.parallel_loop(0, _CHUNK, step=_L, unroll=4)
    def _loop(i):
        t = xin[pl.ds(i, _L)]
        xc = jnp.minimum(jnp.maximum(t, 0.0), 1.0)
        dead = _maxlines(xc, _DEAD_LINES)
        tau = _maxlines(xc, _TAU_LINES)
        delay = dead * _INV_DT
        read = 102.0 - delay
        w = jnp.minimum(jnp.maximum(read - 101.0, 0.0), 1.0)
        w = jnp.where(read >= 102.0, 0.0, w)
        w = jnp.where(delay == 0.0, 1.0, w)
        af = 0.005 / (tau + 0.005)
        xout[pl.ds(i, _L)] = af * (w * t)
    pltpu.sync_copy(xout, out_hbm.at[pl.ds(base, _CHUNK)])


@functools.lru_cache(maxsize=None)
def _build_pam_sc():
    return pl.kernel(
        _sc_body,
        mesh=plsc.VectorSubcoreMesh(core_axis_name="c", subcore_axis_name="s"),
        out_type=jax.ShapeDtypeStruct((_N,), jnp.float32),
        scratch_types=[
            pltpu.VMEM((_CHUNK,), jnp.float32),
            pltpu.VMEM((_CHUNK,), jnp.float32),
        ],
    )


def kernel(target_pressure):
    x = target_pressure.reshape(_N)
    return _build_pam_sc()(x).reshape(_NE, _NCH)


# speedup vs baseline: 1.1969x; 1.0050x over previous
import functools

import jax
import jax.numpy as jnp
from jax import lax
from jax.experimental import pallas as pl
from jax.experimental.pallas import tpu as pltpu
from jax.experimental.pallas import tpu_sc as plsc

_NE, _NCH = 8192, 32
_N = _NE * _NCH
_SC_N = 65536
_TC_N = _N - _SC_N
_NW = 32
_CHUNK = _SC_N // _NW
_L = 16

_DEAD_LINES = ((0.008, 0.0), (0.032, -0.006), (0.36, -0.17), (1.2, -0.8))
_TAU_LINES = ((0.04, 0.02), (0.08, 0.01), (0.12, -0.01), (0.16, -0.04))
_INV_DT = 200.00000447034847


def _maxlines(x, lines):
    (s0, b0), (s1, b1), (s2, b2), (s3, b3) = lines
    return jnp.maximum(
        jnp.maximum(x * s0 + b0, x * s1 + b1),
        jnp.maximum(x * s2 + b2, x * s3 + b3),
    )


def _elem(t):
    xc = jnp.minimum(jnp.maximum(t, 0.0), 1.0)
    dead = _maxlines(xc, _DEAD_LINES)
    tau = _maxlines(xc, _TAU_LINES)
    delay = dead * _INV_DT
    read = 102.0 - delay
    w = jnp.minimum(jnp.maximum(read - 101.0, 0.0), 1.0)
    w = jnp.where(read >= 102.0, 0.0, w)
    w = jnp.where(delay == 0.0, 1.0, w)
    af = 0.005 / (tau + 0.005)
    return af * (w * t)


def _sc_body(x_hbm, out_hbm, xin, xout):
    wid = lax.axis_index("s") * 2 + lax.axis_index("c")
    base = wid * _CHUNK
    pltpu.sync_copy(x_hbm.at[pl.ds(base, _CHUNK)], xin)

    @plsc.parallel_loop(0, _CHUNK, step=_L, unroll=4)
    def _loop(i):
        xout[pl.ds(i, _L)] = _elem(xin[pl.ds(i, _L)])

    pltpu.sync_copy(xout, out_hbm.at[pl.ds(base, _CHUNK)])


@functools.lru_cache(maxsize=None)
def _build_pam_sc():
    return pl.kernel(
        _sc_body,
        mesh=plsc.VectorSubcoreMesh(core_axis_name="c", subcore_axis_name="s"),
        out_type=jax.ShapeDtypeStruct((_SC_N,), jnp.float32),
        scratch_types=[
            pltpu.VMEM((_CHUNK,), jnp.float32),
            pltpu.VMEM((_CHUNK,), jnp.float32),
        ],
    )


def _tc_body(x_ref, o_ref):
    o_ref[...] = _elem(x_ref[...])


_TC_ROWS = _TC_N // 128
_TC_BLOCK = _TC_ROWS // 8


def _tc_call(x):
    return pl.pallas_call(
        _tc_body,
        out_shape=jax.ShapeDtypeStruct((_TC_ROWS, 128), jnp.float32),
        grid=(8,),
        in_specs=[pl.BlockSpec((_TC_BLOCK, 128), lambda i: (i, 0))],
        out_specs=pl.BlockSpec((_TC_BLOCK, 128), lambda i: (i, 0)),
    )(x)


def kernel(target_pressure):
    x = target_pressure.reshape(_N)
    sc_out = _build_pam_sc()(x[:_SC_N])
    tc_out = _tc_call(x[_SC_N:].reshape(_TC_ROWS, 128)).reshape(_TC_N)
    return jnp.concatenate([sc_out, tc_out]).reshape(_NE, _NCH)
